# Initial kernel scaffold; baseline (speedup 1.0000x reference)
import jax
import jax.numpy as jnp
from jax.experimental import pallas as pl


def kernel(x, edge_index, edge_attr, batch, add_des, W0, b0, Wn1, bn1, Wn2, bn2, Wr, bc, Wg, att_src, att_dst, bg, W1, b1, W2, b2):
    raise NotImplementedError("write your pallas kernel here")


# speedup vs baseline: 2.5554x; 2.5554x over previous
import functools

import jax
import jax.numpy as jnp
from jax import lax
from jax.experimental import pallas as pl
from jax.experimental.pallas import tpu as pltpu
from jax.experimental.pallas import tpu_sc as plsc

N = 10000
E = 160000
F = 128
D = 32
G = 64

NC = 2
NS = 16
NW = NC * NS

SUB = 128
IDXROWS = 1280
E_PAD = IDXROWS * SUB
RPW = IDXROWS // NW
KSUB = 10
CPW = RPW // KSUB
CHUNK = SUB * KSUB

N_PAD = 10240
ROWS_PER_SUBCORE = N_PAD // NS

_SC_MESH = dict(core_axis_name="c", subcore_axis_name="s")
_SC_PARAMS = pltpu.CompilerParams(use_tc_tiling_on_sc=False)


def _sc_gather(table, idx2d, width):
    mesh = plsc.VectorSubcoreMesh(**_SC_MESH)

    @functools.partial(
        pl.kernel,
        out_type=jax.ShapeDtypeStruct((E_PAD, width), jnp.float32),
        mesh=mesh,
        compiler_params=_SC_PARAMS,
        scratch_types=[
            pltpu.VMEM((RPW, SUB), jnp.int32),
            pltpu.VMEM((CHUNK, width), jnp.float32),
            pltpu.SemaphoreType.DMA,
        ],
    )
    def k(table_hbm, idx_hbm, out_hbm, idx_v, rows_v, sem):
        cid = lax.axis_index("c")
        sid = lax.axis_index("s")
        wid = sid * NC + cid
        irow0 = wid * RPW

        pltpu.sync_copy(idx_hbm.at[pl.ds(irow0, RPW)], idx_v)
        for c in range(CPW):
            def sub(j, carry):
                pltpu.async_copy(
                    table_hbm.at[idx_v.at[c * KSUB + j]],
                    rows_v.at[pl.ds(j * SUB, SUB)],
                    sem,
                ).wait()
                return carry

            lax.fori_loop(0, KSUB, sub, 0)
            pltpu.sync_copy(
                rows_v, out_hbm.at[pl.ds((irow0 + c * KSUB) * SUB, CHUNK)]
            )

    return k(table, idx2d)


def _sc_scatter(values, idx2d, width):
    mesh = plsc.VectorSubcoreMesh(**_SC_MESH)

    @functools.partial(
        pl.kernel,
        out_type=jax.ShapeDtypeStruct((NC, N_PAD, width), jnp.float32),
        mesh=mesh,
        compiler_params=_SC_PARAMS,
        scratch_types=[
            pltpu.VMEM((RPW, SUB), jnp.int32),
            pltpu.VMEM((CHUNK, width), jnp.float32),
            pltpu.VMEM_SHARED((N_PAD, width), jnp.float32),
            pltpu.SemaphoreType.DMA,
        ],
    )
    def k(vals_hbm, idx_hbm, zero_hbm, out_hbm, idx_v, vals_v, acc_sh, sem):
        cid = lax.axis_index("c")
        sid = lax.axis_index("s")
        wid = sid * NC + cid
        irow0 = wid * RPW
        row0 = sid * ROWS_PER_SUBCORE

        pltpu.sync_copy(zero_hbm, acc_sh.at[pl.ds(row0, ROWS_PER_SUBCORE)])
        pltpu.sync_copy(idx_hbm.at[pl.ds(irow0, RPW)], idx_v)
        plsc.subcore_barrier()

        for c in range(CPW):
            pltpu.sync_copy(
                vals_hbm.at[pl.ds((irow0 + c * KSUB) * SUB, CHUNK)], vals_v
            )

            def sub(j, carry):
                pltpu.sync_copy(
                    vals_v.at[pl.ds(j * SUB, SUB)],
                    acc_sh.at[idx_v.at[c * KSUB + j]],
                    add=True,
                )
                return carry

            lax.fori_loop(0, KSUB, sub, 0)

        plsc.subcore_barrier()
        pltpu.sync_copy(
            acc_sh.at[pl.ds(row0, ROWS_PER_SUBCORE)],
            out_hbm.at[cid, pl.ds(row0, ROWS_PER_SUBCORE)],
        )

    zeros = jnp.zeros((ROWS_PER_SUBCORE, width), jnp.float32)
    return k(values, idx2d, zeros)


def _pre_body(x_ref, w_ref, b_ref, o_ref):
    o_ref[...] = jax.nn.relu(
        jnp.dot(x_ref[...], w_ref[...], preferred_element_type=jnp.float32, precision=lax.Precision.HIGHEST)
        + b_ref[...]
    )


def _dense_pre(x, W0, b0):
    bn = 1000
    return pl.pallas_call(
        _pre_body,
        grid=(N // bn,),
        in_specs=[
            pl.BlockSpec((bn, F), lambda i: (i, 0)),
            pl.BlockSpec((F, D), lambda i: (0, 0)),
            pl.BlockSpec((1, D), lambda i: (0, 0)),
        ],
        out_specs=pl.BlockSpec((bn, D), lambda i: (i, 0)),
        out_shape=jax.ShapeDtypeStruct((N, D), jnp.float32),
    )(x, W0, b0.reshape(1, D))


def _we_body(ea_ref, wn1_ref, bn1_ref, wn2_ref, bn2_ref, o_ref):
    ea = ea_ref[...]
    h = (
        bn1_ref[...]
        + ea[:, 0:1] * wn1_ref[0:1, :]
        + ea[:, 1:2] * wn1_ref[1:2, :]
        + ea[:, 2:3] * wn1_ref[2:3, :]
    )
    h = jax.nn.relu(h)
    o_ref[...] = (
        jnp.dot(h, wn2_ref[...], preferred_element_type=jnp.float32, precision=lax.Precision.HIGHEST)
        + bn2_ref[...]
    )


def _edge_weights(edge_attr, Wn1, bn1, Wn2, bn2):
    be = 640
    return pl.pallas_call(
        _we_body,
        grid=(E // be,),
        in_specs=[
            pl.BlockSpec((be, 3), lambda i: (i, 0)),
            pl.BlockSpec((3, F), lambda i: (0, 0)),
            pl.BlockSpec((1, F), lambda i: (0, 0)),
            pl.BlockSpec((F, D * D), lambda i: (0, 0)),
            pl.BlockSpec((1, D * D), lambda i: (0, 0)),
        ],
        out_specs=pl.BlockSpec((be, D * D), lambda i: (i, 0)),
        out_shape=jax.ShapeDtypeStruct((E, D * D), jnp.float32),
    )(edge_attr, Wn1, bn1.reshape(1, F), Wn2, bn2.reshape(1, D * D))


def _msg_body(nvalid, xs_ref, we_ref, o_ref):
    xs = xs_ref[...]
    acc = xs[:, 0:1] * we_ref[:, 0:D]
    for i in range(1, D):
        acc = acc + xs[:, i : i + 1] * we_ref[:, i * D : (i + 1) * D]
    o_ref[...] = jnp.where(pl.program_id(0) < nvalid, acc, 0.0)


def _messages(xs, we):
    be = 640
    nvalid = E // be
    return pl.pallas_call(
        functools.partial(_msg_body, nvalid),
        grid=(E_PAD // be,),
        in_specs=[
            pl.BlockSpec((be, D), lambda i: (i, 0)),
            pl.BlockSpec((be, D * D), lambda i: (jnp.minimum(i, nvalid - 1), 0)),
        ],
        out_specs=pl.BlockSpec((be, D), lambda i: (i, 0)),
        out_shape=jax.ShapeDtypeStruct((E_PAD, D), jnp.float32),
    )(xs, we)


def _node_body(p_ref, c_ref, z_ref, wr_ref, bc_ref, wg_ref, as_ref, ad_ref,
               hg_ref, aux_ref):
    cnt = jnp.maximum(c_ref[0][:, 0:1] + c_ref[1][:, 0:1], 1.0)
    mean = (p_ref[0] + p_ref[1]) / cnt
    z2 = jax.nn.relu(
        mean
        + jnp.dot(z_ref[...], wr_ref[...], preferred_element_type=jnp.float32, precision=lax.Precision.HIGHEST)
        + bc_ref[...]
    )
    hg = jnp.dot(z2, wg_ref[...], preferred_element_type=jnp.float32, precision=lax.Precision.HIGHEST)
    a_src = jnp.sum(hg * as_ref[...], axis=1, keepdims=True)
    a_dst = jnp.sum(hg * ad_ref[...], axis=1, keepdims=True)
    s = a_src + a_dst
    exl = jnp.exp(jnp.where(s >= 0.0, s, 0.2 * s))
    hg_ref[...] = hg
    aux_ref[...] = jnp.concatenate(
        [a_dst, exl, jnp.zeros((a_dst.shape[0], 14), jnp.float32)], axis=1
    )


def _node_update(part, cnt2, z, Wr, bc, Wg, att_src, att_dst):
    bn = 1000
    return pl.pallas_call(
        _node_body,
        grid=(N // bn,),
        in_specs=[
            pl.BlockSpec((NC, bn, D), lambda i: (0, i, 0)),
            pl.BlockSpec((NC, bn, 16), lambda i: (0, i, 0)),
            pl.BlockSpec((bn, D), lambda i: (i, 0)),
            pl.BlockSpec((D, D), lambda i: (0, 0)),
            pl.BlockSpec((1, D), lambda i: (0, 0)),
            pl.BlockSpec((D, D), lambda i: (0, 0)),
            pl.BlockSpec((1, D), lambda i: (0, 0)),
            pl.BlockSpec((1, D), lambda i: (0, 0)),
        ],
        out_specs=[
            pl.BlockSpec((bn, D), lambda i: (i, 0)),
            pl.BlockSpec((bn, 16), lambda i: (i, 0)),
        ],
        out_shape=[
            jax.ShapeDtypeStruct((N, D), jnp.float32),
            jax.ShapeDtypeStruct((N, 16), jnp.float32),
        ],
    )(part, cnt2, z, Wr, bc.reshape(1, D), Wg,
      att_src.reshape(1, D), att_dst.reshape(1, D))


def _edge_body(nvalid, hsg_ref, adg_ref, as_ref, o_ref):
    hsg = hsg_ref[...]
    a_s = jnp.sum(hsg * as_ref[...], axis=1, keepdims=True)
    s = a_s + adg_ref[:, 0:1]
    ex = jnp.exp(jnp.where(s >= 0.0, s, 0.2 * s))
    out = jnp.concatenate(
        [ex * hsg, ex, jnp.zeros((hsg.shape[0], 15), jnp.float32)], axis=1
    )
    o_ref[...] = jnp.where(pl.program_id(0) < nvalid, out, 0.0)


def _edge_softmax_terms(hsg, adg, att_src):
    be = 1280
    nvalid = E // be
    return pl.pallas_call(
        functools.partial(_edge_body, nvalid),
        grid=(E_PAD // be,),
        in_specs=[
            pl.BlockSpec((be, D), lambda i: (i, 0)),
            pl.BlockSpec((be, 16), lambda i: (i, 0)),
            pl.BlockSpec((1, D), lambda i: (0, 0)),
        ],
        out_specs=pl.BlockSpec((be, 48), lambda i: (i, 0)),
        out_shape=jax.ShapeDtypeStruct((E_PAD, 48), jnp.float32),
    )(hsg, adg, att_src.reshape(1, D))


def _gatfin_body(p_ref, hg_ref, aux_ref, bg_ref, o_ref):
    p = p_ref[0] + p_ref[1]
    exl = aux_ref[:, 1:2]
    hg = hg_ref[...]
    num = p[:, 0:D] + exl * hg
    den = p[:, D : D + 1] + exl
    o_ref[...] = jax.nn.relu(num / den + bg_ref[...])


def _gat_finish(p48, hg, aux, bg):
    bn = 1000
    return pl.pallas_call(
        _gatfin_body,
        grid=(N // bn,),
        in_specs=[
            pl.BlockSpec((NC, bn, 48), lambda i: (0, i, 0)),
            pl.BlockSpec((bn, D), lambda i: (i, 0)),
            pl.BlockSpec((bn, 16), lambda i: (i, 0)),
            pl.BlockSpec((1, D), lambda i: (0, 0)),
        ],
        out_specs=pl.BlockSpec((bn, D), lambda i: (i, 0)),
        out_shape=jax.ShapeDtypeStruct((N, D), jnp.float32),
    )(p48, hg, aux, bg.reshape(1, D))


def _pool_body(m1_ref, m3_ref, m5_ref, b_ref, w1_ref, b1_ref, w2_ref, b2_ref,
               out_ref, acc_ref):
    i = pl.program_id(0)
    ng = pl.num_programs(0)

    @pl.when(i == 0)
    def _():
        acc_ref[...] = jnp.zeros_like(acc_ref)

    o = (m1_ref[...] + m3_ref[...] + m5_ref[...]) * (1.0 / 3.0)
    gids = b_ref[...]
    onehot = (gids == lax.broadcasted_iota(jnp.int32, (1, G), 1)).astype(
        jnp.float32
    )
    psum = lax.dot_general(
        onehot, o, (((0,), (0,)), ((), ())),
        preferred_element_type=jnp.float32, precision=lax.Precision.HIGHEST,
    )
    ones = jnp.ones((o.shape[0], 1), jnp.float32)
    cnt = lax.dot_general(
        onehot, ones, (((0,), (0,)), ((), ())),
        preferred_element_type=jnp.float32, precision=lax.Precision.HIGHEST,
    )
    acc_ref[:, 0:D] += psum
    acc_ref[:, D : D + 1] += cnt

    @pl.when(i == ng - 1)
    def _():
        pooled = acc_ref[:, 0:D] / jnp.maximum(acc_ref[:, D : D + 1], 1.0)
        r = jax.nn.relu(
            jnp.dot(pooled, w1_ref[...], preferred_element_type=jnp.float32, precision=lax.Precision.HIGHEST)
            + b1_ref[...]
        )
        out_ref[...] = (
            jnp.dot(r, w2_ref[...], preferred_element_type=jnp.float32, precision=lax.Precision.HIGHEST)
            + b2_ref[...]
        )


def _pool_mlp(m1, m3, m5, batch2d, W1, b1, W2, b2):
    bn = 1000
    return pl.pallas_call(
        _pool_body,
        grid=(N // bn,),
        in_specs=[
            pl.BlockSpec((bn, D), lambda i: (i, 0)),
            pl.BlockSpec((bn, D), lambda i: (i, 0)),
            pl.BlockSpec((bn, D), lambda i: (i, 0)),
            pl.BlockSpec((bn, 1), lambda i: (i, 0)),
            pl.BlockSpec((D, D), lambda i: (0, 0)),
            pl.BlockSpec((1, D), lambda i: (0, 0)),
            pl.BlockSpec((D, 1), lambda i: (0, 0)),
            pl.BlockSpec((1, 1), lambda i: (0, 0)),
        ],
        out_specs=pl.BlockSpec((G, 1), lambda i: (0, 0)),
        out_shape=jax.ShapeDtypeStruct((G, 1), jnp.float32),
        scratch_shapes=[pltpu.VMEM((G, D + 16), jnp.float32)],
    )(m1, m3, m5, batch2d, W1, b1.reshape(1, D), W2, b2.reshape(1, 1))


def kernel(x, edge_index, edge_attr, batch, add_des, W0, b0, Wn1, bn1, Wn2,
           bn2, Wr, bc, Wg, att_src, att_dst, bg, W1, b1, W2, b2):
    del add_des
    pad = jnp.zeros((E_PAD - E,), jnp.int32)
    src2d = jnp.concatenate([edge_index[0], pad]).reshape(IDXROWS, SUB)
    dst2d = jnp.concatenate([edge_index[1], pad]).reshape(IDXROWS, SUB)

    z0 = _dense_pre(x, W0, b0)
    we = _edge_weights(edge_attr, Wn1, bn1, Wn2, bn2)
    ones16 = jnp.concatenate(
        [jnp.ones((E, 16), jnp.float32), jnp.zeros((E_PAD - E, 16), jnp.float32)]
    )
    cnt2 = _sc_scatter(ones16, dst2d, 16)

    def gnn_round(z):
        zs = _sc_gather(z, src2d, D)
        msg = _messages(zs, we)
        part = _sc_scatter(msg, dst2d, D)
        hg, aux = _node_update(part, cnt2, z, Wr, bc, Wg, att_src, att_dst)
        hsg = _sc_gather(hg, src2d, D)
        adg = _sc_gather(aux, dst2d, 16)
        ew = _edge_softmax_terms(hsg, adg, att_src)
        p48 = _sc_scatter(ew, dst2d, 48)
        return _gat_finish(p48, hg, aux, bg)

    m1 = gnn_round(z0)
    m3 = gnn_round(m1)
    m5 = gnn_round(m3)

    out = _pool_mlp(m1, m3, m5, batch.reshape(N, 1), W1, b1, W2, b2)
    return out.reshape(-1)


# speedup vs baseline: 2.5633x; 1.0031x over previous
import functools

import jax
import jax.numpy as jnp
from jax import lax
from jax.experimental import pallas as pl
from jax.experimental.pallas import tpu as pltpu
from jax.experimental.pallas import tpu_sc as plsc

N = 10000
E = 160000
F = 128
D = 32
G = 64
W48 = 48

NC = 2
NS = 16
NW = NC * NS

SUB = 128
IDXROWS = 1280
E_PAD = IDXROWS * SUB

RPW = IDXROWS // NW
KSUB = 10
CPW = RPW // KSUB
CHUNK = SUB * KSUB

RPC = IDXROWS // NS
KS_S = 8
CPW_S = RPC // KS_S
CHUNK_S = SUB * KS_S

N_PAD = 10240
HALF = N_PAD // NC
NPC = 5248
RPS_C = NPC // NS
BN = 640
NBH = HALF // BN

_SC_MESH = dict(core_axis_name="c", subcore_axis_name="s")
_SC_PARAMS = pltpu.CompilerParams(use_tc_tiling_on_sc=False)
_HI = lax.Precision.HIGHEST


def _sc_gather(table, idx2d, width):
    mesh = plsc.VectorSubcoreMesh(**_SC_MESH)

    @functools.partial(
        pl.kernel,
        out_type=jax.ShapeDtypeStruct((E_PAD, width), jnp.float32),
        mesh=mesh,
        compiler_params=_SC_PARAMS,
        scratch_types=[
            pltpu.VMEM((RPW * SUB,), jnp.int32),
            pltpu.VMEM((2, CHUNK, width), jnp.float32),
            pltpu.SemaphoreType.DMA,
        ],
    )
    def k(table_hbm, idx_hbm, out_hbm, idx_v, rows_v, sem):
        cid = lax.axis_index("c")
        sid = lax.axis_index("s")
        wid = sid * NC + cid
        e0 = wid * RPW * SUB

        pltpu.sync_copy(idx_hbm.at[pl.ds(e0, RPW * SUB)], idx_v)

        cps = {0: pltpu.async_copy(
            table_hbm.at[idx_v.at[pl.ds(0, CHUNK)]], rows_v.at[0], sem)}
        for c in range(CPW):
            if c + 1 < CPW:
                cps[c + 1] = pltpu.async_copy(
                    table_hbm.at[idx_v.at[pl.ds((c + 1) * CHUNK, CHUNK)]],
                    rows_v.at[(c + 1) % 2],
                    sem,
                )
            cps[c].wait()
            pltpu.sync_copy(
                rows_v.at[c % 2], out_hbm.at[pl.ds(e0 + c * CHUNK, CHUNK)]
            )

    return k(table, idx2d)


def _sc_gather2(table_a, idx_a, width_a, table_b, idx_b, width_b):
    mesh = plsc.VectorSubcoreMesh(**_SC_MESH)

    @functools.partial(
        pl.kernel,
        out_type=[
            jax.ShapeDtypeStruct((E_PAD, width_a), jnp.float32),
            jax.ShapeDtypeStruct((E_PAD, width_b), jnp.float32),
        ],
        mesh=mesh,
        compiler_params=_SC_PARAMS,
        scratch_types=[
            pltpu.VMEM((RPW * SUB,), jnp.int32),
            pltpu.VMEM((RPW * SUB,), jnp.int32),
            pltpu.VMEM((CHUNK, width_a), jnp.float32),
            pltpu.VMEM((CHUNK, width_b), jnp.float32),
            pltpu.SemaphoreType.DMA,
            pltpu.SemaphoreType.DMA,
        ],
    )
    def k(ta_hbm, ia_hbm, tb_hbm, ib_hbm, oa_hbm, ob_hbm,
          ia_v, ib_v, ra_v, rb_v, sa, sb):
        cid = lax.axis_index("c")
        sid = lax.axis_index("s")
        wid = sid * NC + cid
        e0 = wid * RPW * SUB

        pltpu.sync_copy(ia_hbm.at[pl.ds(e0, RPW * SUB)], ia_v)
        pltpu.sync_copy(ib_hbm.at[pl.ds(e0, RPW * SUB)], ib_v)

        for c in range(CPW):
            cpa = pltpu.async_copy(
                ta_hbm.at[ia_v.at[pl.ds(c * CHUNK, CHUNK)]], ra_v, sa
            )
            cpb = pltpu.async_copy(
                tb_hbm.at[ib_v.at[pl.ds(c * CHUNK, CHUNK)]], rb_v, sb
            )
            cpa.wait()
            cpb.wait()
            pltpu.sync_copy(
                ra_v, oa_hbm.at[pl.ds(e0 + c * CHUNK, CHUNK)]
            )
            pltpu.sync_copy(
                rb_v, ob_hbm.at[pl.ds(e0 + c * CHUNK, CHUNK)]
            )

    return k(table_a, idx_a, table_b, idx_b)


def _sc_scatter48(values, idxc):
    mesh = plsc.VectorSubcoreMesh(**_SC_MESH)

    @functools.partial(
        pl.kernel,
        out_type=jax.ShapeDtypeStruct((NC, NPC, W48), jnp.float32),
        mesh=mesh,
        compiler_params=_SC_PARAMS,
        scratch_types=[
            pltpu.VMEM((RPC * SUB,), jnp.int32),
            pltpu.VMEM((2, CHUNK_S, W48), jnp.float32),
            pltpu.VMEM_SHARED((NPC, W48), jnp.float32),
            pltpu.SemaphoreType.DMA,
            pltpu.SemaphoreType.DMA,
        ],
    )
    def k(vals_hbm, idx_hbm, zero_hbm, out_hbm, idx_v, vals_v, acc_sh, sem,
          sem_pre):
        cid = lax.axis_index("c")
        sid = lax.axis_index("s")
        e0 = sid * RPC * SUB
        row0 = sid * RPS_C

        pltpu.sync_copy(zero_hbm, acc_sh.at[pl.ds(row0, RPS_C)])
        pltpu.sync_copy(idx_hbm.at[cid, pl.ds(e0, RPC * SUB)], idx_v)
        vcps = {0: pltpu.async_copy(
            vals_hbm.at[pl.ds(e0, CHUNK_S)], vals_v.at[0], sem_pre)}
        plsc.subcore_barrier()

        for c in range(CPW_S):
            if c + 1 < CPW_S:
                vcps[c + 1] = pltpu.async_copy(
                    vals_hbm.at[pl.ds(e0 + (c + 1) * CHUNK_S, CHUNK_S)],
                    vals_v.at[(c + 1) % 2],
                    sem_pre,
                )
            vcps[c].wait()
            pltpu.async_copy(
                vals_v.at[c % 2],
                acc_sh.at[idx_v.at[pl.ds(c * CHUNK_S, CHUNK_S)]],
                sem,
                add=True,
            ).wait()

        plsc.subcore_barrier()
        pltpu.sync_copy(
            acc_sh.at[pl.ds(row0, RPS_C)],
            out_hbm.at[cid, pl.ds(row0, RPS_C)],
        )

    zeros = jnp.zeros((RPS_C, W48), jnp.float32)
    return k(values, idxc.reshape(NC, IDXROWS * SUB), zeros)


def _part_map(i):
    return (i // NBH, i % NBH, 0)


def _pre_body(x_ref, w_ref, b_ref, o_ref):
    o_ref[...] = jax.nn.relu(
        jnp.dot(x_ref[...], w_ref[...], preferred_element_type=jnp.float32,
                precision=_HI)
        + b_ref[...]
    )


def _dense_pre(xp, W0, b0):
    return pl.pallas_call(
        _pre_body,
        grid=(N_PAD // BN,),
        in_specs=[
            pl.BlockSpec((BN, F), lambda i: (i, 0)),
            pl.BlockSpec((F, D), lambda i: (0, 0)),
            pl.BlockSpec((1, D), lambda i: (0, 0)),
        ],
        out_specs=pl.BlockSpec((BN, D), lambda i: (i, 0)),
        out_shape=jax.ShapeDtypeStruct((N_PAD, D), jnp.float32),
    )(xp, W0, b0.reshape(1, D))


def _we_body(ea_ref, wn1_ref, bn1_ref, wn2_ref, bn2_ref, o_ref):
    ea = ea_ref[...]
    h = (
        bn1_ref[...]
        + ea[:, 0:1] * wn1_ref[0:1, :]
        + ea[:, 1:2] * wn1_ref[1:2, :]
        + ea[:, 2:3] * wn1_ref[2:3, :]
    )
    h = jax.nn.relu(h)
    o_ref[...] = (
        jnp.dot(h, wn2_ref[...], preferred_element_type=jnp.float32,
                precision=_HI)
        + bn2_ref[...]
    )


def _edge_weights(edge_attr, Wn1, bn1, Wn2, bn2):
    be = 640
    return pl.pallas_call(
        _we_body,
        grid=(E // be,),
        in_specs=[
            pl.BlockSpec((be, 3), lambda i: (i, 0)),
            pl.BlockSpec((3, F), lambda i: (0, 0)),
            pl.BlockSpec((1, F), lambda i: (0, 0)),
            pl.BlockSpec((F, D * D), lambda i: (0, 0)),
            pl.BlockSpec((1, D * D), lambda i: (0, 0)),
        ],
        out_specs=pl.BlockSpec((be, D * D), lambda i: (i, 0)),
        out_shape=jax.ShapeDtypeStruct((E, D * D), jnp.float32),
    )(edge_attr, Wn1, bn1.reshape(1, F), Wn2, bn2.reshape(1, D * D))


def _msg_body(nvalid, xs_ref, we_ref, r_ref, s_ref, o_ref):
    valid = pl.program_id(0) < nvalid
    spread = jnp.dot(xs_ref[...], r_ref[...],
                     preferred_element_type=jnp.float32, precision=_HI)
    acc = jnp.dot(spread * we_ref[...], s_ref[...],
                  preferred_element_type=jnp.float32, precision=_HI)
    be = acc.shape[0]
    out = jnp.concatenate(
        [acc, jnp.ones((be, 1), jnp.float32),
         jnp.zeros((be, 15), jnp.float32)], axis=1
    )
    o_ref[...] = jnp.where(valid, out, 0.0)


def _messages(xs, we):
    be = 1280
    nvalid = E // be
    ii = jnp.arange(D * D)
    rmat = (ii[None, :] // D == jnp.arange(D)[:, None]).astype(jnp.float32)
    smat = (ii[:, None] % D == jnp.arange(D)[None, :]).astype(jnp.float32)
    return pl.pallas_call(
        functools.partial(_msg_body, nvalid),
        grid=(E_PAD // be,),
        in_specs=[
            pl.BlockSpec((be, D), lambda i: (i, 0)),
            pl.BlockSpec((be, D * D), lambda i: (jnp.minimum(i, nvalid - 1), 0)),
            pl.BlockSpec((D, D * D), lambda i: (0, 0)),
            pl.BlockSpec((D * D, D), lambda i: (0, 0)),
        ],
        out_specs=pl.BlockSpec((be, W48), lambda i: (i, 0)),
        out_shape=jax.ShapeDtypeStruct((E_PAD, W48), jnp.float32),
    )(xs, we, rmat, smat)


def _node_body(p_ref, z_ref, wr_ref, bc_ref, wg_ref, as_ref, ad_ref,
               hg_ref, aux_ref):
    p = p_ref[0]
    cnt = jnp.maximum(p[:, D : D + 1], 1.0)
    mean = p[:, 0:D] / cnt
    z2 = jax.nn.relu(
        mean
        + jnp.dot(z_ref[...], wr_ref[...], preferred_element_type=jnp.float32,
                  precision=_HI)
        + bc_ref[...]
    )
    hg = jnp.dot(z2, wg_ref[...], preferred_element_type=jnp.float32,
                 precision=_HI)
    a_src = jnp.sum(hg * as_ref[...], axis=1, keepdims=True)
    a_dst = jnp.sum(hg * ad_ref[...], axis=1, keepdims=True)
    s = a_src + a_dst
    exl = jnp.exp(jnp.where(s >= 0.0, s, 0.2 * s))
    hg_ref[...] = hg
    aux_ref[...] = jnp.concatenate(
        [a_dst, exl, jnp.zeros((a_dst.shape[0], 14), jnp.float32)], axis=1
    )


def _node_update(part, z, Wr, bc, Wg, att_src, att_dst):
    return pl.pallas_call(
        _node_body,
        grid=(N_PAD // BN,),
        in_specs=[
            pl.BlockSpec((1, BN, W48), _part_map),
            pl.BlockSpec((BN, D), lambda i: (i, 0)),
            pl.BlockSpec((D, D), lambda i: (0, 0)),
            pl.BlockSpec((1, D), lambda i: (0, 0)),
            pl.BlockSpec((D, D), lambda i: (0, 0)),
            pl.BlockSpec((1, D), lambda i: (0, 0)),
            pl.BlockSpec((1, D), lambda i: (0, 0)),
        ],
        out_specs=[
            pl.BlockSpec((BN, D), lambda i: (i, 0)),
            pl.BlockSpec((BN, 16), lambda i: (i, 0)),
        ],
        out_shape=[
            jax.ShapeDtypeStruct((N_PAD, D), jnp.float32),
            jax.ShapeDtypeStruct((N_PAD, 16), jnp.float32),
        ],
    )(part, z, Wr, bc.reshape(1, D), Wg,
      att_src.reshape(1, D), att_dst.reshape(1, D))


def _edge_body(nvalid, hsg_ref, adg_ref, as_ref, o_ref):
    hsg = hsg_ref[...]
    a_s = jnp.sum(hsg * as_ref[...], axis=1, keepdims=True)
    s = a_s + adg_ref[:, 0:1]
    ex = jnp.exp(jnp.where(s >= 0.0, s, 0.2 * s))
    out = jnp.concatenate(
        [ex * hsg, ex, jnp.zeros((hsg.shape[0], 15), jnp.float32)], axis=1
    )
    o_ref[...] = jnp.where(pl.program_id(0) < nvalid, out, 0.0)


def _edge_softmax_terms(hsg, adg, att_src):
    be = 1280
    nvalid = E // be
    return pl.pallas_call(
        functools.partial(_edge_body, nvalid),
        grid=(E_PAD // be,),
        in_specs=[
            pl.BlockSpec((be, D), lambda i: (i, 0)),
            pl.BlockSpec((be, 16), lambda i: (i, 0)),
            pl.BlockSpec((1, D), lambda i: (0, 0)),
        ],
        out_specs=pl.BlockSpec((be, W48), lambda i: (i, 0)),
        out_shape=jax.ShapeDtypeStruct((E_PAD, W48), jnp.float32),
    )(hsg, adg, att_src.reshape(1, D))


def _gatfin_body(p_ref, hg_ref, aux_ref, bg_ref, o_ref):
    p = p_ref[0]
    exl = aux_ref[:, 1:2]
    hg = hg_ref[...]
    num = p[:, 0:D] + exl * hg
    den = p[:, D : D + 1] + exl
    o_ref[...] = jax.nn.relu(num / den + bg_ref[...])


def _gat_finish(p48, hg, aux, bg):
    return pl.pallas_call(
        _gatfin_body,
        grid=(N_PAD // BN,),
        in_specs=[
            pl.BlockSpec((1, BN, W48), _part_map),
            pl.BlockSpec((BN, D), lambda i: (i, 0)),
            pl.BlockSpec((BN, 16), lambda i: (i, 0)),
            pl.BlockSpec((1, D), lambda i: (0, 0)),
        ],
        out_specs=pl.BlockSpec((BN, D), lambda i: (i, 0)),
        out_shape=jax.ShapeDtypeStruct((N_PAD, D), jnp.float32),
    )(p48, hg, aux, bg.reshape(1, D))


def _pool_body(m1_ref, m3_ref, m5_ref, b_ref, w1_ref, b1_ref, w2_ref, b2_ref,
               out_ref, acc_ref):
    i = pl.program_id(0)
    ng = pl.num_programs(0)

    @pl.when(i == 0)
    def _():
        acc_ref[...] = jnp.zeros_like(acc_ref)

    o = (m1_ref[...] + m3_ref[...] + m5_ref[...]) * (1.0 / 3.0)
    gids = b_ref[...]
    onehot = (gids == lax.broadcasted_iota(jnp.int32, (1, G), 1)).astype(
        jnp.float32
    )
    psum = lax.dot_general(
        onehot, o, (((0,), (0,)), ((), ())),
        preferred_element_type=jnp.float32, precision=_HI,
    )
    ones = jnp.ones((o.shape[0], 1), jnp.float32)
    cnt = lax.dot_general(
        onehot, ones, (((0,), (0,)), ((), ())),
        preferred_element_type=jnp.float32, precision=_HI,
    )
    acc_ref[:, 0:D] += psum
    acc_ref[:, D : D + 1] += cnt

    @pl.when(i == ng - 1)
    def _():
        pooled = acc_ref[:, 0:D] / jnp.maximum(acc_ref[:, D : D + 1], 1.0)
        r = jax.nn.relu(
            jnp.dot(pooled, w1_ref[...], preferred_element_type=jnp.float32,
                    precision=_HI)
            + b1_ref[...]
        )
        out_ref[...] = (
            jnp.dot(r, w2_ref[...], preferred_element_type=jnp.float32,
                    precision=_HI)
            + b2_ref[...]
        )


def _pool_mlp(m1, m3, m5, batch2d, W1, b1, W2, b2):
    return pl.pallas_call(
        _pool_body,
        grid=(N_PAD // BN,),
        in_specs=[
            pl.BlockSpec((BN, D), lambda i: (i, 0)),
            pl.BlockSpec((BN, D), lambda i: (i, 0)),
            pl.BlockSpec((BN, D), lambda i: (i, 0)),
            pl.BlockSpec((BN, 1), lambda i: (i, 0)),
            pl.BlockSpec((D, D), lambda i: (0, 0)),
            pl.BlockSpec((1, D), lambda i: (0, 0)),
            pl.BlockSpec((D, 1), lambda i: (0, 0)),
            pl.BlockSpec((1, 1), lambda i: (0, 0)),
        ],
        out_specs=pl.BlockSpec((G, 1), lambda i: (0, 0)),
        out_shape=jax.ShapeDtypeStruct((G, 1), jnp.float32),
        scratch_shapes=[pltpu.VMEM((G, D + 16), jnp.float32)],
    )(m1, m3, m5, batch2d, W1, b1.reshape(1, D), W2, b2.reshape(1, 1))


def kernel(x, edge_index, edge_attr, batch, add_des, W0, b0, Wn1, bn1, Wn2,
           bn2, Wr, bc, Wg, att_src, att_dst, bg, W1, b1, W2, b2):
    del add_des
    pad = jnp.zeros((E_PAD - E,), jnp.int32)
    src = jnp.concatenate([edge_index[0], pad])
    dst = jnp.concatenate([edge_index[1], pad])
    src1d = src
    dst1d = dst
    dst_c0 = jnp.where(dst < HALF, dst, HALF)
    dst_c1 = jnp.where(dst >= HALF, dst - HALF, HALF)
    dstc = jnp.stack([dst_c0, dst_c1]).reshape(NC, IDXROWS, SUB)

    xp = jnp.pad(x, ((0, N_PAD - N), (0, 0)))
    batchp = jnp.pad(batch, (0, N_PAD - N), constant_values=G).reshape(
        N_PAD, 1
    )

    z0 = _dense_pre(xp, W0, b0)
    we = _edge_weights(edge_attr, Wn1, bn1, Wn2, bn2)

    def gnn_round(z):
        zs = _sc_gather(z, src1d, D)
        msg = _messages(zs, we)
        part = _sc_scatter48(msg, dstc)
        hg, aux = _node_update(part, z, Wr, bc, Wg, att_src, att_dst)
        hsg, adg = _sc_gather2(hg, src1d, D, aux, dst1d, 16)
        ew = _edge_softmax_terms(hsg, adg, att_src)
        p48 = _sc_scatter48(ew, dstc)
        return _gat_finish(p48, hg, aux, bg)

    m1 = gnn_round(z0)
    m3 = gnn_round(m1)
    m5 = gnn_round(m3)

    out = _pool_mlp(m1, m3, m5, batchp, W1, b1, W2, b2)
    return out.reshape(-1)


# speedup vs baseline: 4.3582x; 1.7003x over previous
import functools

import jax
import jax.numpy as jnp
from jax import lax
from jax.experimental import pallas as pl
from jax.experimental.pallas import tpu as pltpu
from jax.experimental.pallas import tpu_sc as plsc

N = 10000
E = 160000
F = 128
D = 32
G = 64
W48 = 48

NC = 2
NS = 16
NW = NC * NS

SUB = 128
IDXROWS = 1280
E_PAD = IDXROWS * SUB

RPW = IDXROWS // NW
KSUB = 10
CPW = RPW // KSUB
CHUNK = SUB * KSUB

RPC = IDXROWS // NS
KS_S = 8
CPW_S = RPC // KS_S
CHUNK_S = SUB * KS_S

N_PAD = 10240
HALF = N_PAD // NC
NPC = 5248
RPS_C = NPC // NS
BN = 640
NBH = HALF // BN

_SC_MESH = dict(core_axis_name="c", subcore_axis_name="s")
_SC_PARAMS = pltpu.CompilerParams(use_tc_tiling_on_sc=False)
_HI = lax.Precision.HIGHEST


def _sc_gather(table, idx2d, width):
    mesh = plsc.VectorSubcoreMesh(**_SC_MESH)

    @functools.partial(
        pl.kernel,
        out_type=jax.ShapeDtypeStruct((E_PAD, width), jnp.float32),
        mesh=mesh,
        compiler_params=_SC_PARAMS,
        scratch_types=[
            pltpu.VMEM((RPW * SUB,), jnp.int32),
            pltpu.VMEM((2, CHUNK, width), jnp.float32),
            pltpu.SemaphoreType.DMA,
        ],
    )
    def k(table_hbm, idx_hbm, out_hbm, idx_v, rows_v, sem):
        cid = lax.axis_index("c")
        sid = lax.axis_index("s")
        wid = sid * NC + cid
        e0 = wid * RPW * SUB

        pltpu.sync_copy(idx_hbm.at[pl.ds(e0, RPW * SUB)], idx_v)

        cps = {0: pltpu.async_copy(
            table_hbm.at[idx_v.at[pl.ds(0, CHUNK)]], rows_v.at[0], sem)}
        for c in range(CPW):
            if c + 1 < CPW:
                cps[c + 1] = pltpu.async_copy(
                    table_hbm.at[idx_v.at[pl.ds((c + 1) * CHUNK, CHUNK)]],
                    rows_v.at[(c + 1) % 2],
                    sem,
                )
            cps[c].wait()
            pltpu.sync_copy(
                rows_v.at[c % 2], out_hbm.at[pl.ds(e0 + c * CHUNK, CHUNK)]
            )

    return k(table, idx2d)


def _sc_gather2(table_a, idx_a, width_a, table_b, idx_b, width_b):
    mesh = plsc.VectorSubcoreMesh(**_SC_MESH)

    @functools.partial(
        pl.kernel,
        out_type=[
            jax.ShapeDtypeStruct((E_PAD, width_a), jnp.float32),
            jax.ShapeDtypeStruct((E_PAD, width_b), jnp.float32),
        ],
        mesh=mesh,
        compiler_params=_SC_PARAMS,
        scratch_types=[
            pltpu.VMEM((RPW * SUB,), jnp.int32),
            pltpu.VMEM((RPW * SUB,), jnp.int32),
            pltpu.VMEM((CHUNK, width_a), jnp.float32),
            pltpu.VMEM((CHUNK, width_b), jnp.float32),
            pltpu.SemaphoreType.DMA,
            pltpu.SemaphoreType.DMA,
        ],
    )
    def k(ta_hbm, ia_hbm, tb_hbm, ib_hbm, oa_hbm, ob_hbm,
          ia_v, ib_v, ra_v, rb_v, sa, sb):
        cid = lax.axis_index("c")
        sid = lax.axis_index("s")
        wid = sid * NC + cid
        e0 = wid * RPW * SUB

        pltpu.sync_copy(ia_hbm.at[pl.ds(e0, RPW * SUB)], ia_v)
        pltpu.sync_copy(ib_hbm.at[pl.ds(e0, RPW * SUB)], ib_v)

        for c in range(CPW):
            cpa = pltpu.async_copy(
                ta_hbm.at[ia_v.at[pl.ds(c * CHUNK, CHUNK)]], ra_v, sa
            )
            cpb = pltpu.async_copy(
                tb_hbm.at[ib_v.at[pl.ds(c * CHUNK, CHUNK)]], rb_v, sb
            )
            cpa.wait()
            cpb.wait()
            pltpu.sync_copy(
                ra_v, oa_hbm.at[pl.ds(e0 + c * CHUNK, CHUNK)]
            )
            pltpu.sync_copy(
                rb_v, ob_hbm.at[pl.ds(e0 + c * CHUNK, CHUNK)]
            )

    return k(table_a, idx_a, table_b, idx_b)


def _sc_scatter48(values, idxc):
    mesh = plsc.VectorSubcoreMesh(**_SC_MESH)

    @functools.partial(
        pl.kernel,
        out_type=jax.ShapeDtypeStruct((NC, NPC, W48), jnp.float32),
        mesh=mesh,
        compiler_params=_SC_PARAMS,
        scratch_types=[
            pltpu.VMEM((RPC * SUB,), jnp.int32),
            pltpu.VMEM((2, CHUNK_S, W48), jnp.float32),
            pltpu.VMEM_SHARED((NPC, W48), jnp.float32),
            pltpu.SemaphoreType.DMA,
            pltpu.SemaphoreType.DMA,
        ],
    )
    def k(vals_hbm, idx_hbm, zero_hbm, out_hbm, idx_v, vals_v, acc_sh, sem,
          sem_pre):
        cid = lax.axis_index("c")
        sid = lax.axis_index("s")
        e0 = sid * RPC * SUB
        row0 = sid * RPS_C

        pltpu.sync_copy(zero_hbm, acc_sh.at[pl.ds(row0, RPS_C)])
        pltpu.sync_copy(idx_hbm.at[cid, pl.ds(e0, RPC * SUB)], idx_v)
        vcps = {0: pltpu.async_copy(
            vals_hbm.at[pl.ds(e0, CHUNK_S)], vals_v.at[0], sem_pre)}
        plsc.subcore_barrier()

        for c in range(CPW_S):
            if c + 1 < CPW_S:
                vcps[c + 1] = pltpu.async_copy(
                    vals_hbm.at[pl.ds(e0 + (c + 1) * CHUNK_S, CHUNK_S)],
                    vals_v.at[(c + 1) % 2],
                    sem_pre,
                )
            vcps[c].wait()
            pltpu.async_copy(
                vals_v.at[c % 2],
                acc_sh.at[idx_v.at[pl.ds(c * CHUNK_S, CHUNK_S)]],
                sem,
                add=True,
            ).wait()

        plsc.subcore_barrier()
        pltpu.sync_copy(
            acc_sh.at[pl.ds(row0, RPS_C)],
            out_hbm.at[cid, pl.ds(row0, RPS_C)],
        )

    zeros = jnp.zeros((RPS_C, W48), jnp.float32)
    return k(values, idxc.reshape(NC, IDXROWS * SUB), zeros)


def _part_map(i):
    return (i // NBH, i % NBH, 0)


def _pre_body(x_ref, w_ref, b_ref, o_ref):
    o_ref[...] = jax.nn.relu(
        jnp.dot(x_ref[...], w_ref[...], preferred_element_type=jnp.float32,
                precision=_HI)
        + b_ref[...]
    )


def _dense_pre(xp, W0, b0):
    return pl.pallas_call(
        _pre_body,
        grid=(N_PAD // BN,),
        in_specs=[
            pl.BlockSpec((BN, F), lambda i: (i, 0)),
            pl.BlockSpec((F, D), lambda i: (0, 0)),
            pl.BlockSpec((1, D), lambda i: (0, 0)),
        ],
        out_specs=pl.BlockSpec((BN, D), lambda i: (i, 0)),
        out_shape=jax.ShapeDtypeStruct((N_PAD, D), jnp.float32),
    )(xp, W0, b0.reshape(1, D))


def _we_body(ea_ref, wn1_ref, bn1_ref, wn2_ref, bn2_ref, o_ref):
    ea = ea_ref[...]
    h = (
        bn1_ref[...]
        + ea[:, 0:1] * wn1_ref[0:1, :]
        + ea[:, 1:2] * wn1_ref[1:2, :]
        + ea[:, 2:3] * wn1_ref[2:3, :]
    )
    h = jax.nn.relu(h)
    o_ref[...] = (
        jnp.dot(h, wn2_ref[...], preferred_element_type=jnp.float32,
                precision=_HI)
        + bn2_ref[...]
    )


def _edge_weights(edge_attr, Wn1, bn1, Wn2, bn2):
    be = 640
    return pl.pallas_call(
        _we_body,
        grid=(E // be,),
        in_specs=[
            pl.BlockSpec((be, 3), lambda i: (i, 0)),
            pl.BlockSpec((3, F), lambda i: (0, 0)),
            pl.BlockSpec((1, F), lambda i: (0, 0)),
            pl.BlockSpec((F, D * D), lambda i: (0, 0)),
            pl.BlockSpec((1, D * D), lambda i: (0, 0)),
        ],
        out_specs=pl.BlockSpec((be, D * D), lambda i: (i, 0)),
        out_shape=jax.ShapeDtypeStruct((E, D * D), jnp.float32),
    )(edge_attr, Wn1, bn1.reshape(1, F), Wn2, bn2.reshape(1, D * D))


def _msg_body(nvalid, xs_ref, we_ref, r_ref, s_ref, o_ref):
    valid = pl.program_id(0) < nvalid
    spread = jnp.dot(xs_ref[...], r_ref[...],
                     preferred_element_type=jnp.float32,
                     precision=lax.Precision.DEFAULT)
    acc = jnp.dot(spread * we_ref[...], s_ref[...],
                  preferred_element_type=jnp.float32,
                  precision=lax.Precision.DEFAULT)
    be = acc.shape[0]
    out = jnp.concatenate(
        [acc, jnp.ones((be, 1), jnp.float32),
         jnp.zeros((be, 15), jnp.float32)], axis=1
    )
    o_ref[...] = jnp.where(valid, out, 0.0)


def _messages(xs, we):
    be = 1280
    nvalid = E // be
    ii = jnp.arange(D * D)
    rmat = (ii[None, :] // D == jnp.arange(D)[:, None]).astype(jnp.float32)
    smat = (ii[:, None] % D == jnp.arange(D)[None, :]).astype(jnp.float32)
    return pl.pallas_call(
        functools.partial(_msg_body, nvalid),
        grid=(E_PAD // be,),
        in_specs=[
            pl.BlockSpec((be, D), lambda i: (i, 0)),
            pl.BlockSpec((be, D * D), lambda i: (jnp.minimum(i, nvalid - 1), 0)),
            pl.BlockSpec((D, D * D), lambda i: (0, 0)),
            pl.BlockSpec((D * D, D), lambda i: (0, 0)),
        ],
        out_specs=pl.BlockSpec((be, W48), lambda i: (i, 0)),
        out_shape=jax.ShapeDtypeStruct((E_PAD, W48), jnp.float32),
    )(xs, we, rmat, smat)


def _node_body(p_ref, z_ref, wr_ref, bc_ref, wg_ref, as_ref, ad_ref,
               hg_ref, aux_ref):
    p = p_ref[0]
    cnt = jnp.maximum(p[:, D : D + 1], 1.0)
    mean = p[:, 0:D] / cnt
    z2 = jax.nn.relu(
        mean
        + jnp.dot(z_ref[...], wr_ref[...], preferred_element_type=jnp.float32,
                  precision=_HI)
        + bc_ref[...]
    )
    hg = jnp.dot(z2, wg_ref[...], preferred_element_type=jnp.float32,
                 precision=_HI)
    a_src = jnp.sum(hg * as_ref[...], axis=1, keepdims=True)
    a_dst = jnp.sum(hg * ad_ref[...], axis=1, keepdims=True)
    s = a_src + a_dst
    exl = jnp.exp(jnp.where(s >= 0.0, s, 0.2 * s))
    hg_ref[...] = hg
    aux_ref[...] = jnp.concatenate(
        [a_dst, exl, jnp.zeros((a_dst.shape[0], 14), jnp.float32)], axis=1
    )


def _node_update(part, z, Wr, bc, Wg, att_src, att_dst):
    return pl.pallas_call(
        _node_body,
        grid=(N_PAD // BN,),
        in_specs=[
            pl.BlockSpec((1, BN, W48), _part_map),
            pl.BlockSpec((BN, D), lambda i: (i, 0)),
            pl.BlockSpec((D, D), lambda i: (0, 0)),
            pl.BlockSpec((1, D), lambda i: (0, 0)),
            pl.BlockSpec((D, D), lambda i: (0, 0)),
            pl.BlockSpec((1, D), lambda i: (0, 0)),
            pl.BlockSpec((1, D), lambda i: (0, 0)),
        ],
        out_specs=[
            pl.BlockSpec((BN, D), lambda i: (i, 0)),
            pl.BlockSpec((BN, 16), lambda i: (i, 0)),
        ],
        out_shape=[
            jax.ShapeDtypeStruct((N_PAD, D), jnp.float32),
            jax.ShapeDtypeStruct((N_PAD, 16), jnp.float32),
        ],
    )(part, z, Wr, bc.reshape(1, D), Wg,
      att_src.reshape(1, D), att_dst.reshape(1, D))


def _edge_body(nvalid, hsg_ref, adg_ref, as_ref, o_ref):
    hsg = hsg_ref[...]
    a_s = jnp.sum(hsg * as_ref[...], axis=1, keepdims=True)
    s = a_s + adg_ref[:, 0:1]
    ex = jnp.exp(jnp.where(s >= 0.0, s, 0.2 * s))
    out = jnp.concatenate(
        [ex * hsg, ex, jnp.zeros((hsg.shape[0], 15), jnp.float32)], axis=1
    )
    o_ref[...] = jnp.where(pl.program_id(0) < nvalid, out, 0.0)


def _edge_softmax_terms(hsg, adg, att_src):
    be = 1280
    nvalid = E // be
    return pl.pallas_call(
        functools.partial(_edge_body, nvalid),
        grid=(E_PAD // be,),
        in_specs=[
            pl.BlockSpec((be, D), lambda i: (i, 0)),
            pl.BlockSpec((be, 16), lambda i: (i, 0)),
            pl.BlockSpec((1, D), lambda i: (0, 0)),
        ],
        out_specs=pl.BlockSpec((be, W48), lambda i: (i, 0)),
        out_shape=jax.ShapeDtypeStruct((E_PAD, W48), jnp.float32),
    )(hsg, adg, att_src.reshape(1, D))


def _gatfin_body(p_ref, hg_ref, aux_ref, bg_ref, o_ref):
    p = p_ref[0]
    exl = aux_ref[:, 1:2]
    hg = hg_ref[...]
    num = p[:, 0:D] + exl * hg
    den = p[:, D : D + 1] + exl
    o_ref[...] = jax.nn.relu(num / den + bg_ref[...])


def _gat_finish(p48, hg, aux, bg):
    return pl.pallas_call(
        _gatfin_body,
        grid=(N_PAD // BN,),
        in_specs=[
            pl.BlockSpec((1, BN, W48), _part_map),
            pl.BlockSpec((BN, D), lambda i: (i, 0)),
            pl.BlockSpec((BN, 16), lambda i: (i, 0)),
            pl.BlockSpec((1, D), lambda i: (0, 0)),
        ],
        out_specs=pl.BlockSpec((BN, D), lambda i: (i, 0)),
        out_shape=jax.ShapeDtypeStruct((N_PAD, D), jnp.float32),
    )(p48, hg, aux, bg.reshape(1, D))


def _pool_body(m1_ref, m3_ref, m5_ref, b_ref, w1_ref, b1_ref, w2_ref, b2_ref,
               out_ref, acc_ref):
    i = pl.program_id(0)
    ng = pl.num_programs(0)

    @pl.when(i == 0)
    def _():
        acc_ref[...] = jnp.zeros_like(acc_ref)

    o = (m1_ref[...] + m3_ref[...] + m5_ref[...]) * (1.0 / 3.0)
    gids = b_ref[...]
    onehot = (gids == lax.broadcasted_iota(jnp.int32, (1, G), 1)).astype(
        jnp.float32
    )
    psum = lax.dot_general(
        onehot, o, (((0,), (0,)), ((), ())),
        preferred_element_type=jnp.float32, precision=_HI,
    )
    ones = jnp.ones((o.shape[0], 1), jnp.float32)
    cnt = lax.dot_general(
        onehot, ones, (((0,), (0,)), ((), ())),
        preferred_element_type=jnp.float32, precision=_HI,
    )
    acc_ref[:, 0:D] += psum
    acc_ref[:, D : D + 1] += cnt

    @pl.when(i == ng - 1)
    def _():
        pooled = acc_ref[:, 0:D] / jnp.maximum(acc_ref[:, D : D + 1], 1.0)
        r = jax.nn.relu(
            jnp.dot(pooled, w1_ref[...], preferred_element_type=jnp.float32,
                    precision=_HI)
            + b1_ref[...]
        )
        out_ref[...] = (
            jnp.dot(r, w2_ref[...], preferred_element_type=jnp.float32,
                    precision=_HI)
            + b2_ref[...]
        )


def _pool_mlp(m1, m3, m5, batch2d, W1, b1, W2, b2):
    return pl.pallas_call(
        _pool_body,
        grid=(N_PAD // BN,),
        in_specs=[
            pl.BlockSpec((BN, D), lambda i: (i, 0)),
            pl.BlockSpec((BN, D), lambda i: (i, 0)),
            pl.BlockSpec((BN, D), lambda i: (i, 0)),
            pl.BlockSpec((BN, 1), lambda i: (i, 0)),
            pl.BlockSpec((D, D), lambda i: (0, 0)),
            pl.BlockSpec((1, D), lambda i: (0, 0)),
            pl.BlockSpec((D, 1), lambda i: (0, 0)),
            pl.BlockSpec((1, 1), lambda i: (0, 0)),
        ],
        out_specs=pl.BlockSpec((G, 1), lambda i: (0, 0)),
        out_shape=jax.ShapeDtypeStruct((G, 1), jnp.float32),
        scratch_shapes=[pltpu.VMEM((G, D + 16), jnp.float32)],
    )(m1, m3, m5, batch2d, W1, b1.reshape(1, D), W2, b2.reshape(1, 1))


def kernel(x, edge_index, edge_attr, batch, add_des, W0, b0, Wn1, bn1, Wn2,
           bn2, Wr, bc, Wg, att_src, att_dst, bg, W1, b1, W2, b2):
    del add_des
    pad = jnp.zeros((E_PAD - E,), jnp.int32)
    src = jnp.concatenate([edge_index[0], pad])
    dst = jnp.concatenate([edge_index[1], pad])
    src1d = src
    dst1d = dst
    dst_c0 = jnp.where(dst < HALF, dst, HALF)
    dst_c1 = jnp.where(dst >= HALF, dst - HALF, HALF)
    dstc = jnp.stack([dst_c0, dst_c1]).reshape(NC, IDXROWS, SUB)

    xp = jnp.pad(x, ((0, N_PAD - N), (0, 0)))
    batchp = jnp.pad(batch, (0, N_PAD - N), constant_values=G).reshape(
        N_PAD, 1
    )

    z0 = _dense_pre(xp, W0, b0)
    we = _edge_weights(edge_attr, Wn1, bn1, Wn2, bn2)

    def gnn_round(z):
        zs = _sc_gather(z, src1d, D)
        msg = _messages(zs, we)
        part = _sc_scatter48(msg, dstc)
        hg, aux = _node_update(part, z, Wr, bc, Wg, att_src, att_dst)
        hsg, adg = _sc_gather2(hg, src1d, D, aux, dst1d, 16)
        ew = _edge_softmax_terms(hsg, adg, att_src)
        p48 = _sc_scatter48(ew, dstc)
        return _gat_finish(p48, hg, aux, bg)

    m1 = gnn_round(z0)
    m3 = gnn_round(m1)
    m5 = gnn_round(m3)

    out = _pool_mlp(m1, m3, m5, batchp, W1, b1, W2, b2)
    return out.reshape(-1)
